# baseline (speedup 1.0000x reference)
import functools

import jax
import jax.numpy as jnp
from jax import lax
from jax.experimental import pallas as pl
from jax.experimental.pallas import tpu as pltpu
from jax.experimental.pallas import tpu_sc as plsc

NC = 2
NS = 16
LANES = 16
TILES = NC * NS


def _make_deg_rs(n_pad, rows_per_core):
    npt = n_pad // NS
    cpt = rows_per_core // NS

    mesh = plsc.VectorSubcoreMesh(core_axis_name="c", subcore_axis_name="s",
                                  num_cores=NC, num_subcores=NS)

    @functools.partial(
        pl.kernel,
        out_type=(jax.ShapeDtypeStruct((n_pad,), jnp.float32),
                  jax.ShapeDtypeStruct((n_pad,), jnp.float32)),
        mesh=mesh,
        scratch_types=[
            pltpu.VMEM_SHARED((n_pad,), jnp.float32),
            pltpu.VMEM((cpt, 128), jnp.int32),
            pltpu.VMEM((cpt, 128), jnp.float32),
            pltpu.VMEM((npt,), jnp.float32),
            pltpu.SemaphoreType.DMA,
        ],
    )
    def deg_rs(rc_hbm, v_hbm, degr_hbm, degc_hbm, degacc, idxb, vb, lbuf,
               sem):
        cid = lax.axis_index("c")
        sid = lax.axis_index("s")
        for k in range(npt // LANES):
            lbuf[pl.ds(k * LANES, LANES)] = jnp.zeros((LANES,), jnp.float32)
        pltpu.sync_copy(lbuf, degacc.at[pl.ds(sid * npt, npt)])
        plsc.subcore_barrier()
        pltpu.sync_copy(rc_hbm.at[cid, pl.ds(sid * cpt, cpt)], idxb)
        pltpu.sync_copy(v_hbm.at[pl.ds(sid * cpt, cpt)], vb)

        def fire(j, carry):
            pltpu.async_copy(vb.at[j], degacc.at[idxb.at[j]], sem, add=True)
            return carry

        lax.fori_loop(0, cpt, fire, 0)

        def drain(j, carry):
            pltpu.make_async_copy(vb.at[j], degacc.at[idxb.at[j]], sem).wait()
            return carry

        lax.fori_loop(0, cpt, drain, 0)
        plsc.subcore_barrier()

        @pl.when(cid == 0)
        def _w0():
            pltpu.sync_copy(degacc.at[pl.ds(sid * npt, npt)],
                            degr_hbm.at[pl.ds(sid * npt, npt)])

        @pl.when(cid == 1)
        def _w1():
            pltpu.sync_copy(degacc.at[pl.ds(sid * npt, npt)],
                            degc_hbm.at[pl.ds(sid * npt, npt)])

    return deg_rs


def _make_scatter(compute_norm, acc_n, n_pad, d, rows):
    hd = d // 2
    npt = acc_n // NS
    cpt = rows // TILES
    S = 12
    RING = 3
    assert cpt % S == 0 and npt % 8 == 0
    n_st = cpt // S

    mesh = plsc.VectorSubcoreMesh(core_axis_name="c", subcore_axis_name="s",
                                  num_cores=NC, num_subcores=NS)

    part_t = jax.ShapeDtypeStruct((NC, 2, n_pad, hd), jnp.float32)
    norm_t = jax.ShapeDtypeStruct((TILES, n_st, S, 128), jnp.float32)
    if compute_norm:
        out_type = (part_t, norm_t)
    else:
        out_type = part_t

    scratch = [
        pltpu.VMEM_SHARED((acc_n, hd), jnp.float32),
        pltpu.VMEM_SHARED((acc_n, hd), jnp.float32),
        pltpu.VMEM((S, 128), jnp.int32),
        pltpu.VMEM((S, 128), jnp.int32),
        pltpu.VMEM((S, 128), jnp.float32),
        pltpu.VMEM((128, hd), jnp.float32),
        pltpu.VMEM((128, hd), jnp.float32),
        pltpu.VMEM((128, hd), jnp.float32),
        pltpu.VMEM((S, 128), jnp.float32),
        pltpu.VMEM((S, 128), jnp.float32),
        pltpu.SemaphoreType.DMA,
        pltpu.SemaphoreType.DMA,
        pltpu.SemaphoreType.DMA,
        pltpu.SemaphoreType.DMA,
        pltpu.SemaphoreType.DMA,
        pltpu.SemaphoreType.DMA,
    ]

    @functools.partial(
        pl.kernel, out_type=out_type, mesh=mesh, scratch_types=scratch,
        compiler_params=pltpu.CompilerParams(use_tc_tiling_on_sc=False))
    def scatter(*refs):
        if compute_norm:
            (xw0, xw1, rc, v4, rsr, rsc, part, norm_hbm, acc, xws,
             irs, ics, nbs, rb0, rb1, rb2, ga, gb,
             g0, g1, g2, s0, s1, s2) = refs
        else:
            (xw0, xw1, rc, norm_hbm, part, acc, xws,
             irs, ics, nbs, rb0, rb1, rb2, ga, gb,
             g0, g1, g2, s0, s1, s2) = refs
        cid = lax.axis_index("c")
        sid = lax.axis_index("s")
        gid = cid * NS + sid
        rbufs = (rb0, rb1, rb2)
        gsems = (g0, g1, g2)
        ssems = (s0, s1, s2)

        if compute_norm:
            def nstage(st, carry):
                pltpu.sync_copy(rc.at[0, gid, st], irs)
                pltpu.sync_copy(rc.at[1, gid, st], ics)
                pltpu.sync_copy(v4.at[gid, st], nbs)

                def gfire(j, c2):
                    pltpu.async_copy(rsr.at[irs.at[j]], ga.at[j], g0)
                    pltpu.async_copy(rsc.at[ics.at[j]], gb.at[j], g1)
                    return c2

                lax.fori_loop(0, S, gfire, 0)

                def gdrain(j, c2):
                    pltpu.make_async_copy(rsr.at[irs.at[j]], ga.at[j],
                                          g0).wait()
                    pltpu.make_async_copy(rsc.at[ics.at[j]], gb.at[j],
                                          g1).wait()
                    return c2

                lax.fori_loop(0, S, gdrain, 0)

                def nrow(j, c2):
                    for g in range(128 // LANES):
                        sl = pl.ds(g * LANES, LANES)
                        nbs[j, sl] = ga[j, sl] * gb[j, sl] * nbs[j, sl]
                    return c2

                lax.fori_loop(0, S, nrow, 0)
                pltpu.sync_copy(nbs, norm_hbm.at[gid, st])
                return carry

            lax.fori_loop(0, n_st, nstage, 0)

        def scale(jl, k):
            rb = rbufs[k]

            def do16(r16, carry):
                nv = nbs[jl, pl.ds(r16 * LANES, LANES)]
                for l in range(LANES):
                    sp = nv.at[jnp.full((LANES,), l, jnp.int32)].get(
                        mode="promise_in_bounds")
                    row = r16 * LANES + l
                    for g in range(hd // LANES):
                        sl = pl.ds(g * LANES, LANES)
                        rb[row, sl] = rb[row, sl] * sp
                return carry

            lax.fori_loop(0, 128 // LANES, do16, 0)

        for half in range(2):
            xwh = (xw0, xw1)[half]
            pltpu.sync_copy(xwh.at[pl.ds(sid * npt, npt)],
                            xws.at[pl.ds(sid * npt, npt)])

            def zrow(i, carry):
                for g in range(hd // LANES):
                    rb0[i, pl.ds(g * LANES, LANES)] = jnp.zeros(
                        (LANES,), jnp.float32)
                return carry

            lax.fori_loop(0, 128, zrow, 0)
            zoff = 0
            while zoff < npt:
                zc = min(128, npt - zoff)
                pltpu.sync_copy(rb0.at[pl.ds(0, zc)],
                                acc.at[pl.ds(sid * npt + zoff, zc)])
                zoff += zc
            plsc.subcore_barrier()

            def stage(st, carry):
                @pl.when(st > 0)
                def _wps():
                    kl = (S - 1) % RING
                    pltpu.make_async_copy(rbufs[kl], acc.at[ics.at[S - 1]],
                                          ssems[kl]).wait()

                pltpu.sync_copy(rc.at[0, gid, st], irs)
                pltpu.sync_copy(rc.at[1, gid, st], ics)
                pltpu.sync_copy(norm_hbm.at[gid, st], nbs)
                for jl in range(2):
                    pltpu.async_copy(xws.at[irs.at[jl]], rbufs[jl],
                                     gsems[jl])
                for jl in range(S):
                    k = jl % RING
                    kp = (jl - 1) % RING
                    pltpu.make_async_copy(xws.at[irs.at[jl]], rbufs[k],
                                          gsems[k]).wait()
                    scale(jl, k)
                    pltpu.async_copy(rbufs[k], acc.at[ics.at[jl]],
                                     ssems[k], add=True)
                    if jl > 0:
                        pltpu.make_async_copy(rbufs[kp],
                                              acc.at[ics.at[jl - 1]],
                                              ssems[kp]).wait()
                    if jl + 2 < S:
                        pltpu.async_copy(xws.at[irs.at[jl + 2]],
                                         rbufs[(jl + 2) % RING],
                                         gsems[(jl + 2) % RING])
                return carry

            lax.fori_loop(0, n_st, stage, 0)
            klast = (S - 1) % RING
            pltpu.make_async_copy(rbufs[klast], acc.at[ics.at[S - 1]],
                                  ssems[klast]).wait()
            plsc.subcore_barrier()

            pltpu.sync_copy(acc.at[pl.ds(sid * npt, npt)],
                            part.at[cid, half, pl.ds(sid * npt, npt)])

    return scatter


def _rsqrt2(dr, dc):
    m, d = dr.shape

    def body(a_r, b_r, oa_r, ob_r):
        oa_r[...] = lax.rsqrt(a_r[...])
        ob_r[...] = lax.rsqrt(b_r[...])

    full = lambda i: (0, 0)
    return pl.pallas_call(
        body,
        grid=(1,),
        in_specs=[pl.BlockSpec((m, d), full), pl.BlockSpec((m, d), full)],
        out_specs=[pl.BlockSpec((m, d), full), pl.BlockSpec((m, d), full)],
        out_shape=[jax.ShapeDtypeStruct((m, d), jnp.float32),
                   jax.ShapeDtypeStruct((m, d), jnp.float32)],
    )(dr, dc)


def _mm(x, w, bm):
    n, d = x.shape

    def body(x_r, w_r, o_r):
        o_r[...] = jnp.dot(x_r[...], w_r[...],
                           preferred_element_type=jnp.float32)

    return pl.pallas_call(
        body,
        grid=(n // bm,),
        in_specs=[pl.BlockSpec((bm, d), lambda i: (i, 0)),
                  pl.BlockSpec((d, w.shape[1]), lambda i: (0, 0))],
        out_specs=pl.BlockSpec((bm, w.shape[1]), lambda i: (i, 0)),
        out_shape=jax.ShapeDtypeStruct((n, w.shape[1]), jnp.float32),
    )(x, w)


def _combine_relu_mm(p00, p01, p10, p11, b, w, bm):
    n, hd = p00.shape
    d = 2 * hd

    def body(p00_r, p01_r, p10_r, p11_r, b_r, w_r, o_r):
        p = jnp.concatenate([p00_r[...] + p10_r[...],
                             p01_r[...] + p11_r[...]], axis=1)
        h = jnp.maximum(p + b_r[...], 0.0)
        o_r[...] = jnp.dot(h, w_r[...], preferred_element_type=jnp.float32)

    row_h = lambda i: (i, 0)
    fix = lambda i: (0, 0)
    return pl.pallas_call(
        body,
        grid=(n // bm,),
        in_specs=[pl.BlockSpec((bm, hd), row_h), pl.BlockSpec((bm, hd), row_h),
                  pl.BlockSpec((bm, hd), row_h), pl.BlockSpec((bm, hd), row_h),
                  pl.BlockSpec((1, d), fix),
                  pl.BlockSpec((d, d), fix)],
        out_specs=pl.BlockSpec((bm, d), row_h),
        out_shape=jax.ShapeDtypeStruct((n, d), jnp.float32),
    )(p00, p01, p10, p11, b, w)


def _head(q00, q01, q10, q11, b2, p1w, pb1, p2w, pb2, bm):
    n, hd = q00.shape
    d = 2 * hd

    def body(q00_r, q01_r, q10_r, q11_r, b2_r, p1_r, pb1_r, p2_r, pb2_r,
             emb_o, z_o):
        q = jnp.concatenate([q00_r[...] + q10_r[...],
                             q01_r[...] + q11_r[...]], axis=1)
        emb = q + b2_r[...]
        emb_o[...] = emb
        t = jnp.maximum(
            jnp.dot(emb, p1_r[...], preferred_element_type=jnp.float32)
            + pb1_r[...], 0.0)
        z_o[...] = (jnp.dot(t, p2_r[...], preferred_element_type=jnp.float32)
                    + pb2_r[...])

    row = lambda i: (i, 0)
    fix = lambda i: (0, 0)
    return pl.pallas_call(
        body,
        grid=(n // bm,),
        in_specs=[pl.BlockSpec((bm, hd), row), pl.BlockSpec((bm, hd), row),
                  pl.BlockSpec((bm, hd), row), pl.BlockSpec((bm, hd), row),
                  pl.BlockSpec((1, d), fix), pl.BlockSpec((d, d), fix),
                  pl.BlockSpec((1, d), fix), pl.BlockSpec((d, d), fix),
                  pl.BlockSpec((1, d), fix)],
        out_specs=[pl.BlockSpec((bm, d), row), pl.BlockSpec((bm, d), row)],
        out_shape=[jax.ShapeDtypeStruct((n, d), jnp.float32),
                   jax.ShapeDtypeStruct((n, d), jnp.float32)],
    )(q00, q01, q10, q11, b2, p1w, pb1, p2w, pb2)


def kernel(x_, edge_index, edge_val, W1, b1, W2, b2, P1, pb1, P2, pb2):
    n, d = x_.shape
    e = edge_val.shape[0]

    e_ext = e + n
    cpt = -(-e_ext // (TILES * 128))
    cpt = -(-cpt // 12) * 12
    e_pad = TILES * 128 * cpt
    rows = e_pad // 128
    n_pad = -(-n // (NS * 128)) * (NS * 128)
    acc_n = -(-n // (NS * 8)) * (NS * 8)
    bm = 512

    loop = jnp.arange(n, dtype=jnp.int32)
    zi = jnp.zeros((e_pad - e_ext,), jnp.int32)
    r_all = jnp.concatenate([edge_index[0], loop, zi])
    c_all = jnp.concatenate([edge_index[1], loop, zi])
    v_all = jnp.concatenate([edge_val, jnp.ones((n,), jnp.float32),
                             jnp.zeros((e_pad - e_ext,), jnp.float32)])
    n_st = cpt // 12
    rc = jnp.stack([r_all, c_all]).reshape(2, rows, 128)
    rc5 = rc.reshape(2, TILES, n_st, 12, 128)
    v3 = v_all.reshape(rows, 128)
    v4 = v3.reshape(TILES, n_st, 12, 128)
    xp = jnp.pad(x_, ((0, n_pad - n), (0, 0)))

    hd = d // 2
    degr, degc = _make_deg_rs(n_pad, rows)(rc, v3)
    rsr2, rsc2 = _rsqrt2(degr.reshape(-1, 128), degc.reshape(-1, 128))
    rsr, rsc = rsr2.reshape(-1), rsc2.reshape(-1)
    xw = _mm(xp, W1, bm)
    part1, norm = _make_scatter(True, acc_n, n_pad, d, rows)(
        xw[:, :hd], xw[:, hd:], rc5, v4, rsr, rsc)
    hw = _combine_relu_mm(part1[0, 0], part1[0, 1], part1[1, 0], part1[1, 1],
                          b1.reshape(1, d), W2, bm)
    part2 = _make_scatter(False, acc_n, n_pad, d, rows)(
        hw[:, :hd], hw[:, hd:], rc5, norm)
    emb, z = _head(part2[0, 0], part2[0, 1], part2[1, 0], part2[1, 1],
                   b2.reshape(1, d), P1, pb1.reshape(1, d), P2,
                   pb2.reshape(1, d), bm)
    return emb[:n], z[:n]


# speedup vs baseline: 1.0739x; 1.0739x over previous
import functools

import jax
import jax.numpy as jnp
from jax import lax
from jax.experimental import pallas as pl
from jax.experimental.pallas import tpu as pltpu
from jax.experimental.pallas import tpu_sc as plsc

NC = 2
NS = 16
LANES = 16
TILES = NC * NS


def _make_deg(n_pad, rows_per_core):
    npt = n_pad // NS
    cpt = rows_per_core // NS

    mesh = plsc.VectorSubcoreMesh(core_axis_name="c", subcore_axis_name="s",
                                  num_cores=NC, num_subcores=NS)

    @functools.partial(
        pl.kernel,
        out_type=(jax.ShapeDtypeStruct((n_pad,), jnp.float32),
                  jax.ShapeDtypeStruct((n_pad,), jnp.float32)),
        mesh=mesh,
        scratch_types=[
            pltpu.VMEM_SHARED((n_pad,), jnp.float32),
            pltpu.VMEM((cpt, 128), jnp.int32),
            pltpu.VMEM((cpt, 128), jnp.float32),
            pltpu.VMEM((npt,), jnp.float32),
            pltpu.SemaphoreType.DMA,
        ],
    )
    def deg(rc_hbm, v_hbm, degr_hbm, degc_hbm, degacc, idxb, vb, lbuf, sem):
        cid = lax.axis_index("c")
        sid = lax.axis_index("s")
        for k in range(npt // LANES):
            lbuf[pl.ds(k * LANES, LANES)] = jnp.zeros((LANES,), jnp.float32)
        pltpu.sync_copy(lbuf, degacc.at[pl.ds(sid * npt, npt)])
        plsc.subcore_barrier()
        pltpu.sync_copy(rc_hbm.at[cid, pl.ds(sid * cpt, cpt)], idxb)
        pltpu.sync_copy(v_hbm.at[pl.ds(sid * cpt, cpt)], vb)

        def fire(j, carry):
            pltpu.async_copy(vb.at[j], degacc.at[idxb.at[j]], sem, add=True)
            return carry

        lax.fori_loop(0, cpt, fire, 0)

        def drain(j, carry):
            pltpu.make_async_copy(vb.at[j], degacc.at[idxb.at[j]], sem).wait()
            return carry

        lax.fori_loop(0, cpt, drain, 0)
        plsc.subcore_barrier()

        @pl.when(cid == 0)
        def _w0():
            pltpu.sync_copy(degacc.at[pl.ds(sid * npt, npt)],
                            degr_hbm.at[pl.ds(sid * npt, npt)])

        @pl.when(cid == 1)
        def _w1():
            pltpu.sync_copy(degacc.at[pl.ds(sid * npt, npt)],
                            degc_hbm.at[pl.ds(sid * npt, npt)])

    return deg


def _make_scatter(acc_n, n_pad, d, rows):
    hd = d // 2
    npt = acc_n // NS
    cpt = rows // TILES
    S = 12
    RING = 3
    assert cpt % S == 0 and npt % 8 == 0
    n_st = cpt // S

    mesh = plsc.VectorSubcoreMesh(core_axis_name="c", subcore_axis_name="s",
                                  num_cores=NC, num_subcores=NS)

    scratch = [
        pltpu.VMEM_SHARED((acc_n, hd), jnp.float32),
        pltpu.VMEM_SHARED((acc_n, hd), jnp.float32),
        pltpu.VMEM((S, 128), jnp.int32),
        pltpu.VMEM((S, 128), jnp.int32),
        pltpu.VMEM((S, 128), jnp.float32),
        pltpu.VMEM((128, hd), jnp.float32),
        pltpu.VMEM((128, hd), jnp.float32),
        pltpu.VMEM((128, hd), jnp.float32),
        pltpu.SemaphoreType.DMA,
        pltpu.SemaphoreType.DMA,
        pltpu.SemaphoreType.DMA,
        pltpu.SemaphoreType.DMA,
        pltpu.SemaphoreType.DMA,
        pltpu.SemaphoreType.DMA,
    ]

    @functools.partial(
        pl.kernel,
        out_type=jax.ShapeDtypeStruct((NC, 2, n_pad, hd), jnp.float32),
        mesh=mesh, scratch_types=scratch,
        compiler_params=pltpu.CompilerParams(use_tc_tiling_on_sc=False))
    def scatter(xw0, xw1, rc, vv, part, acc, xws, irs, ics, nbs,
                rb0, rb1, rb2, g0, g1, g2, s0, s1, s2):
        cid = lax.axis_index("c")
        sid = lax.axis_index("s")
        gid = cid * NS + sid
        rbufs = (rb0, rb1, rb2)
        gsems = (g0, g1, g2)
        ssems = (s0, s1, s2)

        def scale(jl, k):
            rb = rbufs[k]

            def do16(r16, carry):
                nv = nbs[jl, pl.ds(r16 * LANES, LANES)]
                for l in range(LANES):
                    sp = nv.at[jnp.full((LANES,), l, jnp.int32)].get(
                        mode="promise_in_bounds")
                    row = r16 * LANES + l
                    for g in range(hd // LANES):
                        sl = pl.ds(g * LANES, LANES)
                        rb[row, sl] = rb[row, sl] * sp
                return carry

            lax.fori_loop(0, 128 // LANES, do16, 0)

        for half in range(2):
            xwh = (xw0, xw1)[half]
            pltpu.sync_copy(xwh.at[pl.ds(sid * npt, npt)],
                            xws.at[pl.ds(sid * npt, npt)])

            def zrow(i, carry):
                for g in range(hd // LANES):
                    rb0[i, pl.ds(g * LANES, LANES)] = jnp.zeros(
                        (LANES,), jnp.float32)
                return carry

            lax.fori_loop(0, 128, zrow, 0)
            zoff = 0
            while zoff < npt:
                zc = min(128, npt - zoff)
                pltpu.sync_copy(rb0.at[pl.ds(0, zc)],
                                acc.at[pl.ds(sid * npt + zoff, zc)])
                zoff += zc
            plsc.subcore_barrier()

            def stage(st, carry):
                @pl.when(st > 0)
                def _wps():
                    kl = (S - 1) % RING
                    pltpu.make_async_copy(rbufs[kl], acc.at[ics.at[S - 1]],
                                          ssems[kl]).wait()

                pltpu.sync_copy(rc.at[0, gid, st], irs)
                pltpu.sync_copy(rc.at[1, gid, st], ics)
                pltpu.sync_copy(vv.at[gid, st], nbs)
                for jl in range(2):
                    pltpu.async_copy(xws.at[irs.at[jl]], rbufs[jl],
                                     gsems[jl])
                for jl in range(S):
                    k = jl % RING
                    kp = (jl - 1) % RING
                    pltpu.make_async_copy(xws.at[irs.at[jl]], rbufs[k],
                                          gsems[k]).wait()
                    scale(jl, k)
                    pltpu.async_copy(rbufs[k], acc.at[ics.at[jl]],
                                     ssems[k], add=True)
                    if jl > 0:
                        pltpu.make_async_copy(rbufs[kp],
                                              acc.at[ics.at[jl - 1]],
                                              ssems[kp]).wait()
                    if jl + 2 < S:
                        pltpu.async_copy(xws.at[irs.at[jl + 2]],
                                         rbufs[(jl + 2) % RING],
                                         gsems[(jl + 2) % RING])
                return carry

            lax.fori_loop(0, n_st, stage, 0)
            klast = (S - 1) % RING
            pltpu.make_async_copy(rbufs[klast], acc.at[ics.at[S - 1]],
                                  ssems[klast]).wait()
            plsc.subcore_barrier()

            pltpu.sync_copy(acc.at[pl.ds(sid * npt, npt)],
                            part.at[cid, half, pl.ds(sid * npt, npt)])

    return scatter


def _mm_rowscale(x, w, degr, bm):
    n, d = x.shape

    def body(x_r, w_r, dg_r, o_r):
        o_r[...] = jnp.dot(x_r[...], w_r[...],
                           preferred_element_type=jnp.float32) \
            * lax.rsqrt(dg_r[...])

    return pl.pallas_call(
        body,
        grid=(n // bm,),
        in_specs=[pl.BlockSpec((bm, d), lambda i: (i, 0)),
                  pl.BlockSpec((d, w.shape[1]), lambda i: (0, 0)),
                  pl.BlockSpec((bm, 1), lambda i: (i, 0))],
        out_specs=pl.BlockSpec((bm, w.shape[1]), lambda i: (i, 0)),
        out_shape=jax.ShapeDtypeStruct((n, w.shape[1]), jnp.float32),
    )(x, w, degr)


def _combine_relu_mm(p00, p01, p10, p11, degc, degr, b, w, bm):
    n, hd = p00.shape
    d = 2 * hd

    def body(p00_r, p01_r, p10_r, p11_r, dc_r, dr_r, b_r, w_r, o_r):
        p = jnp.concatenate([p00_r[...] + p10_r[...],
                             p01_r[...] + p11_r[...]], axis=1)
        h = jnp.maximum(p * lax.rsqrt(dc_r[...]) + b_r[...], 0.0)
        o_r[...] = jnp.dot(h, w_r[...], preferred_element_type=jnp.float32) \
            * lax.rsqrt(dr_r[...])

    row_h = lambda i: (i, 0)
    row_1 = lambda i: (i, 0)
    fix = lambda i: (0, 0)
    return pl.pallas_call(
        body,
        grid=(n // bm,),
        in_specs=[pl.BlockSpec((bm, hd), row_h), pl.BlockSpec((bm, hd), row_h),
                  pl.BlockSpec((bm, hd), row_h), pl.BlockSpec((bm, hd), row_h),
                  pl.BlockSpec((bm, 1), row_1), pl.BlockSpec((bm, 1), row_1),
                  pl.BlockSpec((1, d), fix),
                  pl.BlockSpec((d, d), fix)],
        out_specs=pl.BlockSpec((bm, d), row_h),
        out_shape=jax.ShapeDtypeStruct((n, d), jnp.float32),
    )(p00, p01, p10, p11, degc, degr, b, w)


def _head(q00, q01, q10, q11, degc, b2, p1w, pb1, p2w, pb2, bm):
    n, hd = q00.shape
    d = 2 * hd

    def body(q00_r, q01_r, q10_r, q11_r, dc_r, b2_r, p1_r, pb1_r, p2_r,
             pb2_r, emb_o, z_o):
        q = jnp.concatenate([q00_r[...] + q10_r[...],
                             q01_r[...] + q11_r[...]], axis=1)
        emb = q * lax.rsqrt(dc_r[...]) + b2_r[...]
        emb_o[...] = emb
        t = jnp.maximum(
            jnp.dot(emb, p1_r[...], preferred_element_type=jnp.float32)
            + pb1_r[...], 0.0)
        z_o[...] = (jnp.dot(t, p2_r[...], preferred_element_type=jnp.float32)
                    + pb2_r[...])

    row = lambda i: (i, 0)
    fix = lambda i: (0, 0)
    return pl.pallas_call(
        body,
        grid=(n // bm,),
        in_specs=[pl.BlockSpec((bm, hd), row), pl.BlockSpec((bm, hd), row),
                  pl.BlockSpec((bm, hd), row), pl.BlockSpec((bm, hd), row),
                  pl.BlockSpec((bm, 1), row),
                  pl.BlockSpec((1, d), fix), pl.BlockSpec((d, d), fix),
                  pl.BlockSpec((1, d), fix), pl.BlockSpec((d, d), fix),
                  pl.BlockSpec((1, d), fix)],
        out_specs=[pl.BlockSpec((bm, d), row), pl.BlockSpec((bm, d), row)],
        out_shape=[jax.ShapeDtypeStruct((n, d), jnp.float32),
                   jax.ShapeDtypeStruct((n, d), jnp.float32)],
    )(q00, q01, q10, q11, degc, b2, p1w, pb1, p2w, pb2)


def kernel(x_, edge_index, edge_val, W1, b1, W2, b2, P1, pb1, P2, pb2):
    n, d = x_.shape
    e = edge_val.shape[0]

    e_ext = e + n
    cpt = -(-e_ext // (TILES * 128))
    cpt = -(-cpt // 12) * 12
    e_pad = TILES * 128 * cpt
    rows = e_pad // 128
    n_pad = -(-n // (NS * 128)) * (NS * 128)
    acc_n = -(-n // (NS * 8)) * (NS * 8)
    bm = 512

    loop = jnp.arange(n, dtype=jnp.int32)
    zi = jnp.zeros((e_pad - e_ext,), jnp.int32)
    r_all = jnp.concatenate([edge_index[0], loop, zi])
    c_all = jnp.concatenate([edge_index[1], loop, zi])
    v_all = jnp.concatenate([edge_val, jnp.ones((n,), jnp.float32),
                             jnp.zeros((e_pad - e_ext,), jnp.float32)])
    n_st = cpt // 12
    rc = jnp.stack([r_all, c_all]).reshape(2, rows, 128)
    rc5 = rc.reshape(2, TILES, n_st, 12, 128)
    v3 = v_all.reshape(rows, 128)
    v4 = v3.reshape(TILES, n_st, 12, 128)
    xp = jnp.pad(x_, ((0, n_pad - n), (0, 0)))

    hd = d // 2
    degr, degc = _make_deg(n_pad, rows)(rc, v3)
    dgr2 = degr.reshape(n_pad, 1)
    dgc2 = degc.reshape(n_pad, 1)
    xw = _mm_rowscale(xp, W1, dgr2, bm)
    part1 = _make_scatter(acc_n, n_pad, d, rows)(
        xw[:, :hd], xw[:, hd:], rc5, v4)
    hw = _combine_relu_mm(part1[0, 0], part1[0, 1], part1[1, 0], part1[1, 1],
                          dgc2[:n_pad], dgr2[:n_pad], b1.reshape(1, d), W2,
                          bm)
    part2 = _make_scatter(acc_n, n_pad, d, rows)(
        hw[:, :hd], hw[:, hd:], rc5, v4)
    emb, z = _head(part2[0, 0], part2[0, 1], part2[1, 0], part2[1, 1],
                   dgc2[:n_pad], b2.reshape(1, d), P1, pb1.reshape(1, d), P2,
                   pb2.reshape(1, d), bm)
    return emb[:n], z[:n]


# speedup vs baseline: 1.0980x; 1.0225x over previous
import functools

import jax
import jax.numpy as jnp
from jax import lax
from jax.experimental import pallas as pl
from jax.experimental.pallas import tpu as pltpu
from jax.experimental.pallas import tpu_sc as plsc

NC = 2
NS = 16
LANES = 16
TILES = NC * NS
SCHUNK = 21


def _make_deg(n_pad, rows_per_core):
    npt = n_pad // NS
    cpt = rows_per_core // NS

    mesh = plsc.VectorSubcoreMesh(core_axis_name="c", subcore_axis_name="s",
                                  num_cores=NC, num_subcores=NS)

    @functools.partial(
        pl.kernel,
        out_type=(jax.ShapeDtypeStruct((n_pad,), jnp.float32),
                  jax.ShapeDtypeStruct((n_pad,), jnp.float32)),
        mesh=mesh,
        scratch_types=[
            pltpu.VMEM_SHARED((n_pad,), jnp.float32),
            pltpu.VMEM((cpt, 128), jnp.int32),
            pltpu.VMEM((cpt, 128), jnp.float32),
            pltpu.VMEM((npt,), jnp.float32),
            pltpu.SemaphoreType.DMA,
        ],
    )
    def deg(rc_hbm, v_hbm, degr_hbm, degc_hbm, degacc, idxb, vb, lbuf, sem):
        cid = lax.axis_index("c")
        sid = lax.axis_index("s")
        for k in range(npt // LANES):
            lbuf[pl.ds(k * LANES, LANES)] = jnp.zeros((LANES,), jnp.float32)
        pltpu.sync_copy(lbuf, degacc.at[pl.ds(sid * npt, npt)])
        plsc.subcore_barrier()
        pltpu.sync_copy(rc_hbm.at[cid, pl.ds(sid * cpt, cpt)], idxb)
        pltpu.sync_copy(v_hbm.at[pl.ds(sid * cpt, cpt)], vb)

        def fire(j, carry):
            pltpu.async_copy(vb.at[j], degacc.at[idxb.at[j]], sem, add=True)
            return carry

        lax.fori_loop(0, cpt, fire, 0)

        def drain(j, carry):
            pltpu.make_async_copy(vb.at[j], degacc.at[idxb.at[j]], sem).wait()
            return carry

        lax.fori_loop(0, cpt, drain, 0)
        plsc.subcore_barrier()

        @pl.when(cid == 0)
        def _w0():
            pltpu.sync_copy(degacc.at[pl.ds(sid * npt, npt)],
                            degr_hbm.at[pl.ds(sid * npt, npt)])

        @pl.when(cid == 1)
        def _w1():
            pltpu.sync_copy(degacc.at[pl.ds(sid * npt, npt)],
                            degc_hbm.at[pl.ds(sid * npt, npt)])

    return deg


def _make_scatter(acc_n, n_pad, d, rows):
    hd = d // 2
    npt = acc_n // NS
    cpt = rows // TILES
    S = SCHUNK
    RING = 3
    assert cpt % S == 0 and npt % 8 == 0
    n_st = cpt // S

    mesh = plsc.VectorSubcoreMesh(core_axis_name="c", subcore_axis_name="s",
                                  num_cores=NC, num_subcores=NS)

    scratch = [
        pltpu.VMEM_SHARED((acc_n, hd), jnp.float32),
        pltpu.VMEM_SHARED((acc_n, hd), jnp.float32),
        pltpu.VMEM((S, 128), jnp.int32),
        pltpu.VMEM((S, 128), jnp.int32),
        pltpu.VMEM((S, 128), jnp.float32),
        pltpu.VMEM((128, hd), jnp.float32),
        pltpu.VMEM((128, hd), jnp.float32),
        pltpu.VMEM((128, hd), jnp.float32),
        pltpu.SemaphoreType.DMA,
        pltpu.SemaphoreType.DMA,
        pltpu.SemaphoreType.DMA,
        pltpu.SemaphoreType.DMA,
        pltpu.SemaphoreType.DMA,
        pltpu.SemaphoreType.DMA,
    ]

    @functools.partial(
        pl.kernel,
        out_type=jax.ShapeDtypeStruct((NC, 2, n_pad, hd), jnp.float32),
        mesh=mesh, scratch_types=scratch,
        compiler_params=pltpu.CompilerParams(use_tc_tiling_on_sc=False))
    def scatter(xw0, xw1, rc, vv, part, acc, xws, irs, ics, nbs,
                rb0, rb1, rb2, g0, g1, g2, s0, s1, s2):
        cid = lax.axis_index("c")
        sid = lax.axis_index("s")
        gid = cid * NS + sid
        rbufs = (rb0, rb1, rb2)
        gsems = (g0, g1, g2)
        ssems = (s0, s1, s2)

        def scale(jl, k):
            rb = rbufs[k]

            def do16(r16, carry):
                nv = nbs[jl, pl.ds(r16 * LANES, LANES)]
                for l in range(LANES):
                    sp = nv.at[jnp.full((LANES,), l, jnp.int32)].get(
                        mode="promise_in_bounds")
                    row = r16 * LANES + l
                    for g in range(hd // LANES):
                        sl = pl.ds(g * LANES, LANES)
                        rb[row, sl] = rb[row, sl] * sp
                return carry

            lax.fori_loop(0, 128 // LANES, do16, 0)

        for half in range(2):
            xwh = (xw0, xw1)[half]
            pltpu.sync_copy(xwh.at[pl.ds(sid * npt, npt)],
                            xws.at[pl.ds(sid * npt, npt)])

            def zrow(i, carry):
                for g in range(hd // LANES):
                    rb0[i, pl.ds(g * LANES, LANES)] = jnp.zeros(
                        (LANES,), jnp.float32)
                return carry

            lax.fori_loop(0, 128, zrow, 0)
            zoff = 0
            while zoff < npt:
                zc = min(128, npt - zoff)
                pltpu.sync_copy(rb0.at[pl.ds(0, zc)],
                                acc.at[pl.ds(sid * npt + zoff, zc)])
                zoff += zc
            plsc.subcore_barrier()

            def stage(st, carry):
                @pl.when(st > 0)
                def _wps():
                    kl = (S - 1) % RING
                    pltpu.make_async_copy(rbufs[kl], acc.at[ics.at[S - 1]],
                                          ssems[kl]).wait()

                pltpu.sync_copy(rc.at[0, gid, st], irs)
                pltpu.sync_copy(rc.at[1, gid, st], ics)
                pltpu.sync_copy(vv.at[gid, st], nbs)
                for jl in range(2):
                    pltpu.async_copy(xws.at[irs.at[jl]], rbufs[jl],
                                     gsems[jl])
                for jl in range(S):
                    k = jl % RING
                    kp = (jl - 1) % RING
                    pltpu.make_async_copy(xws.at[irs.at[jl]], rbufs[k],
                                          gsems[k]).wait()
                    scale(jl, k)
                    pltpu.async_copy(rbufs[k], acc.at[ics.at[jl]],
                                     ssems[k], add=True)
                    if jl > 0:
                        pltpu.make_async_copy(rbufs[kp],
                                              acc.at[ics.at[jl - 1]],
                                              ssems[kp]).wait()
                    if jl + 2 < S:
                        pltpu.async_copy(xws.at[irs.at[jl + 2]],
                                         rbufs[(jl + 2) % RING],
                                         gsems[(jl + 2) % RING])
                return carry

            lax.fori_loop(0, n_st, stage, 0)
            klast = (S - 1) % RING
            pltpu.make_async_copy(rbufs[klast], acc.at[ics.at[S - 1]],
                                  ssems[klast]).wait()
            plsc.subcore_barrier()

            pltpu.sync_copy(acc.at[pl.ds(sid * npt, npt)],
                            part.at[cid, half, pl.ds(sid * npt, npt)])

    return scatter


def _mm_rowscale(x, w, degr, bm):
    n, d = x.shape

    def body(x_r, w_r, dg_r, o_r):
        o_r[...] = jnp.dot(x_r[...], w_r[...],
                           preferred_element_type=jnp.float32) \
            * lax.rsqrt(dg_r[...])

    return pl.pallas_call(
        body,
        grid=(n // bm,),
        in_specs=[pl.BlockSpec((bm, d), lambda i: (i, 0)),
                  pl.BlockSpec((d, w.shape[1]), lambda i: (0, 0)),
                  pl.BlockSpec((bm, 1), lambda i: (i, 0))],
        out_specs=pl.BlockSpec((bm, w.shape[1]), lambda i: (i, 0)),
        out_shape=jax.ShapeDtypeStruct((n, w.shape[1]), jnp.float32),
    )(x, w, degr)


def _combine_relu_mm(p00, p01, p10, p11, degc, degr, b, w, bm):
    n, hd = p00.shape
    d = 2 * hd

    def body(p00_r, p01_r, p10_r, p11_r, dc_r, dr_r, b_r, w_r, o_r):
        p = jnp.concatenate([p00_r[...] + p10_r[...],
                             p01_r[...] + p11_r[...]], axis=1)
        h = jnp.maximum(p * lax.rsqrt(dc_r[...]) + b_r[...], 0.0)
        o_r[...] = jnp.dot(h, w_r[...], preferred_element_type=jnp.float32) \
            * lax.rsqrt(dr_r[...])

    row_h = lambda i: (i, 0)
    row_1 = lambda i: (i, 0)
    fix = lambda i: (0, 0)
    return pl.pallas_call(
        body,
        grid=(n // bm,),
        in_specs=[pl.BlockSpec((bm, hd), row_h), pl.BlockSpec((bm, hd), row_h),
                  pl.BlockSpec((bm, hd), row_h), pl.BlockSpec((bm, hd), row_h),
                  pl.BlockSpec((bm, 1), row_1), pl.BlockSpec((bm, 1), row_1),
                  pl.BlockSpec((1, d), fix),
                  pl.BlockSpec((d, d), fix)],
        out_specs=pl.BlockSpec((bm, d), row_h),
        out_shape=jax.ShapeDtypeStruct((n, d), jnp.float32),
    )(p00, p01, p10, p11, degc, degr, b, w)


def _head(q00, q01, q10, q11, degc, b2, p1w, pb1, p2w, pb2, bm):
    n, hd = q00.shape
    d = 2 * hd

    def body(q00_r, q01_r, q10_r, q11_r, dc_r, b2_r, p1_r, pb1_r, p2_r,
             pb2_r, emb_o, z_o):
        q = jnp.concatenate([q00_r[...] + q10_r[...],
                             q01_r[...] + q11_r[...]], axis=1)
        emb = q * lax.rsqrt(dc_r[...]) + b2_r[...]
        emb_o[...] = emb
        t = jnp.maximum(
            jnp.dot(emb, p1_r[...], preferred_element_type=jnp.float32)
            + pb1_r[...], 0.0)
        z_o[...] = (jnp.dot(t, p2_r[...], preferred_element_type=jnp.float32)
                    + pb2_r[...])

    row = lambda i: (i, 0)
    fix = lambda i: (0, 0)
    return pl.pallas_call(
        body,
        grid=(n // bm,),
        in_specs=[pl.BlockSpec((bm, hd), row), pl.BlockSpec((bm, hd), row),
                  pl.BlockSpec((bm, hd), row), pl.BlockSpec((bm, hd), row),
                  pl.BlockSpec((bm, 1), row),
                  pl.BlockSpec((1, d), fix), pl.BlockSpec((d, d), fix),
                  pl.BlockSpec((1, d), fix), pl.BlockSpec((d, d), fix),
                  pl.BlockSpec((1, d), fix)],
        out_specs=[pl.BlockSpec((bm, d), row), pl.BlockSpec((bm, d), row)],
        out_shape=[jax.ShapeDtypeStruct((n, d), jnp.float32),
                   jax.ShapeDtypeStruct((n, d), jnp.float32)],
    )(q00, q01, q10, q11, degc, b2, p1w, pb1, p2w, pb2)


def kernel(x_, edge_index, edge_val, W1, b1, W2, b2, P1, pb1, P2, pb2):
    n, d = x_.shape
    e = edge_val.shape[0]

    e_ext = e + n
    cpt = -(-e_ext // (TILES * 128))
    cpt = -(-cpt // SCHUNK) * SCHUNK
    e_pad = TILES * 128 * cpt
    rows = e_pad // 128
    n_pad = -(-n // (NS * 128)) * (NS * 128)
    acc_n = -(-n // (NS * 8)) * (NS * 8)
    bm = 512

    loop = jnp.arange(n, dtype=jnp.int32)
    zi = jnp.zeros((e_pad - e_ext,), jnp.int32)
    r_all = jnp.concatenate([edge_index[0], loop, zi])
    c_all = jnp.concatenate([edge_index[1], loop, zi])
    v_all = jnp.concatenate([edge_val, jnp.ones((n,), jnp.float32),
                             jnp.zeros((e_pad - e_ext,), jnp.float32)])
    n_st = cpt // SCHUNK
    rc = jnp.stack([r_all, c_all]).reshape(2, rows, 128)
    rc5 = rc.reshape(2, TILES, n_st, SCHUNK, 128)
    v3 = v_all.reshape(rows, 128)
    v4 = v3.reshape(TILES, n_st, SCHUNK, 128)
    xp = jnp.pad(x_, ((0, n_pad - n), (0, 0)))

    hd = d // 2
    degr, degc = _make_deg(n_pad, rows)(rc, v3)
    dgr2 = degr.reshape(n_pad, 1)
    dgc2 = degc.reshape(n_pad, 1)
    xw = _mm_rowscale(xp, W1, dgr2, bm)
    part1 = _make_scatter(acc_n, n_pad, d, rows)(
        xw[:, :hd], xw[:, hd:], rc5, v4)
    hw = _combine_relu_mm(part1[0, 0], part1[0, 1], part1[1, 0], part1[1, 1],
                          dgc2[:n_pad], dgr2[:n_pad], b1.reshape(1, d), W2,
                          bm)
    part2 = _make_scatter(acc_n, n_pad, d, rows)(
        hw[:, :hd], hw[:, hd:], rc5, v4)
    emb, z = _head(part2[0, 0], part2[0, 1], part2[1, 0], part2[1, 1],
                   dgc2[:n_pad], b2.reshape(1, d), P1, pb1.reshape(1, d), P2,
                   pb2.reshape(1, d), bm)
    return emb[:n], z[:n]
